# Initial kernel scaffold; baseline (speedup 1.0000x reference)
import jax
import jax.numpy as jnp
from jax.experimental import pallas as pl


def kernel(x, gw1, gb1, glg, glb, gw2, gb2, ew1, eb1, el1g, el1b, ew2, eb2, el2g, el2b, ew3, eb3, opw, opb):
    raise NotImplementedError("write your pallas kernel here")


# speedup vs baseline: 4.1916x; 4.1916x over previous
import functools

import jax
import jax.numpy as jnp
from jax.experimental import pallas as pl
from jax.experimental.pallas import tpu as pltpu

_E = 8
_K = 2


def _layernorm(h, g, b):
    mu = jnp.mean(h, axis=-1, keepdims=True)
    v = jnp.mean((h - mu) ** 2, axis=-1, keepdims=True)
    return (h - mu) * jax.lax.rsqrt(v + 1e-5) * g + b


def _gelu(h):
    return h * 0.5 * (1.0 + jax.lax.erf(h * 0.7071067811865476))


def _moe_dense_body(x_ref, gw1_ref, gb1_ref, glg_ref, glb_ref, gw2_ref, gb2_ref,
                    ew1_ref, eb1_ref, el1g_ref, el1b_ref, ew2_ref, eb2_ref,
                    el2g_ref, el2b_ref, ew3_ref, eb3_ref, opw_ref, opb_ref,
                    out_ref, gw_ref, tki_ref, tkw_ref, usage_ref, *, n_tokens):
    xb = x_ref[...]
    h = jnp.dot(xb, gw1_ref[...], preferred_element_type=jnp.float32) + gb1_ref[...]
    h = _gelu(_layernorm(h, glg_ref[...], glb_ref[...]))
    logits = jnp.dot(h, gw2_ref[...], preferred_element_type=jnp.float32) + gb2_ref[...]
    m = jnp.max(logits, axis=-1, keepdims=True)
    ex = jnp.exp(logits - m)
    gate_w = ex / jnp.sum(ex, axis=-1, keepdims=True)
    gw_ref[...] = gate_w

    lane = jax.lax.broadcasted_iota(jnp.int32, gate_w.shape, 1)
    w0 = jnp.max(gate_w, axis=-1, keepdims=True)
    i0 = jnp.min(jnp.where(gate_w == w0, lane, _E), axis=-1, keepdims=True)
    masked = jnp.where(lane == i0, -1.0, gate_w)
    w1 = jnp.max(masked, axis=-1, keepdims=True)
    i1 = jnp.min(jnp.where(masked == w1, lane, _E), axis=-1, keepdims=True)
    s = w0 + w1
    wn0 = w0 / s
    wn1 = w1 / s
    tki_ref[...] = jnp.concatenate([i0, i1], axis=-1)
    tkw_ref[...] = jnp.concatenate([wn0, wn1], axis=-1)

    acc = jnp.zeros((xb.shape[0], out_ref.shape[-1]), jnp.float32)
    for i in range(_E):
        wsel = jnp.where(i0 == i, wn0, 0.0) + jnp.where(i1 == i, wn1, 0.0)
        h1 = jnp.dot(xb, ew1_ref[i], preferred_element_type=jnp.float32) + eb1_ref[i]
        h1 = _gelu(_layernorm(h1, el1g_ref[i], el1b_ref[i]))
        h2 = jnp.dot(h1, ew2_ref[i], preferred_element_type=jnp.float32) + eb2_ref[i]
        h2 = _gelu(_layernorm(h2, el2g_ref[i], el2b_ref[i]))
        eo = jnp.dot(h2, ew3_ref[i], preferred_element_type=jnp.float32) + eb3_ref[i]
        acc = acc + eo * wsel
    out_ref[...] = jnp.dot(acc, opw_ref[...], preferred_element_type=jnp.float32) + opb_ref[...]

    @pl.when(pl.program_id(0) == 0)
    def _():
        usage_ref[...] = jnp.zeros_like(usage_ref)

    usage_ref[...] += jnp.sum(gate_w, axis=0, keepdims=True) * (1.0 / n_tokens)


def kernel(x, gw1, gb1, glg, glb, gw2, gb2, ew1, eb1, el1g, el1b, ew2, eb2,
           el2g, el2b, ew3, eb3, opw, opb, *, interpret=False):
    n, d = x.shape
    h_dim = gw1.shape[1]
    o_dim = ew3.shape[2]
    bt = min(512, n)
    grid = n // bt

    full = lambda shape: pl.BlockSpec(shape, lambda i: (0,) * len(shape))

    out_shapes = (
        jax.ShapeDtypeStruct((n, o_dim), jnp.float32),
        jax.ShapeDtypeStruct((n, _E), jnp.float32),
        jax.ShapeDtypeStruct((n, _K), jnp.int32),
        jax.ShapeDtypeStruct((n, _K), jnp.float32),
        jax.ShapeDtypeStruct((1, _E), jnp.float32),
    )
    out_specs = (
        pl.BlockSpec((bt, o_dim), lambda i: (i, 0)),
        pl.BlockSpec((bt, _E), lambda i: (i, 0)),
        pl.BlockSpec((bt, _K), lambda i: (i, 0)),
        pl.BlockSpec((bt, _K), lambda i: (i, 0)),
        full((1, _E)),
    )
    in_specs = [
        pl.BlockSpec((bt, d), lambda i: (i, 0)),
        full((d, h_dim)),
        full((1, h_dim)), full((1, h_dim)), full((1, h_dim)),
        full((h_dim, _E)), full((1, _E)),
        full((_E, d, h_dim)), full((_E, 1, h_dim)),
        full((_E, 1, h_dim)), full((_E, 1, h_dim)),
        full((_E, h_dim, h_dim)), full((_E, 1, h_dim)),
        full((_E, 1, h_dim)), full((_E, 1, h_dim)),
        full((_E, h_dim, o_dim)), full((_E, 1, o_dim)),
        full((o_dim, o_dim)), full((1, o_dim)),
    ]

    out, gate_w, tki, tkw, usage = pl.pallas_call(
        functools.partial(_moe_dense_body, n_tokens=n),
        grid=(grid,),
        in_specs=in_specs,
        out_specs=out_specs,
        out_shape=out_shapes,
        compiler_params=pltpu.CompilerParams(
            dimension_semantics=("arbitrary",),
        ),
        interpret=interpret,
    )(
        x, gw1,
        gb1.reshape(1, h_dim), glg.reshape(1, h_dim), glb.reshape(1, h_dim),
        gw2, gb2.reshape(1, _E),
        ew1, eb1.reshape(_E, 1, h_dim),
        el1g.reshape(_E, 1, h_dim), el1b.reshape(_E, 1, h_dim),
        ew2, eb2.reshape(_E, 1, h_dim),
        el2g.reshape(_E, 1, h_dim), el2b.reshape(_E, 1, h_dim),
        ew3, eb3.reshape(_E, 1, o_dim),
        opw, opb.reshape(1, o_dim),
    )
    return (out, gate_w, tki, tkw, usage.reshape(_E))
